# baseline (speedup 1.0000x reference)
import jax
import jax.numpy as jnp
from jax import lax
from jax.experimental import pallas as pl
from jax.experimental.pallas import tpu as pltpu
from jax.experimental.pallas import tpu_sc as plsc

_N = 10000
_E = 320000
_FIN = 128
_H = 8
_D = 16
_HD = 128
_P = 2
_C = 16

_NPAD = 10240
_NC = 2
_NS = 16
_NW = _NC * _NS
_EPW = 10240
_EPAD = _EPW * _NW
_CHA = 128
_NCA = _EPW // _CHA
_CHS = 80
_NCS = _EPW // _CHS
_RPT = _NPAD // _NS
_BLK = 1024
_NBLK = _NPAD // _BLK

_F32 = jnp.float32
_SC_PARAMS = pltpu.CompilerParams(use_tc_tiling_on_sc=False)


def _mesh():
    return plsc.VectorSubcoreMesh(core_axis_name="c", subcore_axis_name="s",
                                  num_cores=_NC, num_subcores=_NS)


def _prep_body(x_ref, w_ref, al_ref, ar_ref, g_ref, feat_ref, l_ref, r_ref):
    f = jnp.dot(x_ref[...], w_ref[0], preferred_element_type=_F32)
    feat_ref[0] = jnp.dot(f, g_ref[...], preferred_element_type=_F32)
    l_ref[0] = jnp.dot(f, al_ref[0], preferred_element_type=_F32)
    r_ref[0] = jnp.dot(f, ar_ref[0], preferred_element_type=_F32)


def _prep(xpad, W_gat, AL, AR, G):
    return pl.pallas_call(
        _prep_body,
        grid=(_P, _NBLK),
        in_specs=[
            pl.BlockSpec((_BLK, _FIN), lambda p, i: (i, 0)),
            pl.BlockSpec((1, _FIN, _HD), lambda p, i: (p, 0, 0)),
            pl.BlockSpec((1, _HD, 16), lambda p, i: (p, 0, 0)),
            pl.BlockSpec((1, _HD, 16), lambda p, i: (p, 0, 0)),
            pl.BlockSpec((_HD, _HD), lambda p, i: (0, 0)),
        ],
        out_specs=[
            pl.BlockSpec((1, _BLK, _HD), lambda p, i: (p, i, 0)),
            pl.BlockSpec((1, _BLK, 16), lambda p, i: (p, i, 0)),
            pl.BlockSpec((1, _BLK, 16), lambda p, i: (p, i, 0)),
        ],
        out_shape=[
            jax.ShapeDtypeStruct((_P, _NPAD, _HD), _F32),
            jax.ShapeDtypeStruct((_P, _NPAD, 16), _F32),
            jax.ShapeDtypeStruct((_P, _NPAD, 16), _F32),
        ],
    )(xpad, W_gat, AL, AR, G)


def _attn_body(l0, r0, l1, r1, src0, dst0, src1, dst1, s_out0, s_out1,
               ex0, ex1,
               isb, idb,
               lv0, rv0, ev0,
               lv1, rv1, ev1,
               zb, sacc0, sacc1,
               semg0, semg1, semw0, semw1, semsc0, semsc1):
    cid = lax.axis_index("c")
    sid = lax.axis_index("s")
    wid = sid * _NC + cid
    SETS = ((lv0, rv0, ev0, semg0, semw0, semsc0),
            (lv1, rv1, ev1, semg1, semw1, semsc1))

    def _zrow(i, c):
        zb[i, :] = jnp.zeros((16,), _F32)
        return c

    lax.fori_loop(0, _RPT, _zrow, 0)
    pltpu.sync_copy(zb, sacc0.at[pl.ds(sid * _RPT, _RPT)])
    pltpu.sync_copy(zb, sacc1.at[pl.ds(sid * _RPT, _RPT)])
    plsc.subcore_barrier()

    for p in range(_P):
        lt = (l0, l1)[p]
        rt = (r0, r1)[p]
        se = (src0, src1)[p]
        de = (dst0, dst1)[p]
        sacc = (sacc0, sacc1)[p]
        ext = (ex0, ex1)[p]

        pltpu.sync_copy(se.at[pl.ds(wid * _NCA, _NCA)], isb)
        pltpu.sync_copy(de.at[pl.ds(wid * _NCA, _NCA)], idb)

        def _off(k):
            return wid * _EPW + k * _CHA

        def _issue(k, b):
            lv, rv, ev, semg, semw, semsc = SETS[b]
            pltpu.async_copy(lt.at[isb.at[k]], lv, semg)
            pltpu.async_copy(rt.at[idb.at[k]], rv, semg)

        def _wait_g(k, b):
            lv, rv, ev, semg, semw, semsc = SETS[b]
            pltpu.make_async_copy(lt.at[isb.at[k]], lv, semg).wait()
            pltpu.make_async_copy(rt.at[idb.at[k]], rv, semg).wait()

        def _compute(b):
            lv, rv, ev, semg, semw, semsc = SETS[b]

            def _edge(i, cc):
                e = lv[i, :] + rv[i, :]
                e = jnp.where(e > 0, e, 0.2 * e)
                ev[i, :] = jnp.exp(e)
                return cc

            lax.fori_loop(0, _CHA, _edge, 0, unroll=4)

        def _scatter(k, b):
            lv, rv, ev, semg, semw, semsc = SETS[b]
            pltpu.async_copy(ev, ext.at[pl.ds(_off(k), _CHA)], semw)
            return pltpu.async_copy(ev, sacc.at[idb.at[k]], semsc, add=True)

        def _wait_w(k, b):
            lv, rv, ev, semg, semw, semsc = SETS[b]
            pltpu.make_async_copy(ev, ext.at[pl.ds(_off(k), _CHA)], semw).wait()

        _issue(0, 0)
        _issue(1, 1)
        _wait_g(0, 0)
        _compute(0)
        c0 = _scatter(0, 0)
        _wait_g(1, 1)
        _compute(1)
        c1 = _scatter(1, 1)
        c0.wait()
        _issue(2, 0)
        c1.wait()
        _issue(3, 1)

        def _pipe(t, c):
            _wait_g(2 * t, 0)
            _wait_w(2 * t - 2, 0)
            _compute(0)
            s0 = _scatter(2 * t, 0)
            _wait_g(2 * t + 1, 1)
            _wait_w(2 * t - 1, 1)
            _compute(1)
            s1 = _scatter(2 * t + 1, 1)
            s0.wait()
            _issue(2 * t + 2, 0)
            s1.wait()
            _issue(2 * t + 3, 1)
            return c

        lax.fori_loop(1, _NCA // 2 - 1, _pipe, 0)
        _wait_g(_NCA - 2, 0)
        _wait_w(_NCA - 4, 0)
        _compute(0)
        s0 = _scatter(_NCA - 2, 0)
        _wait_g(_NCA - 1, 1)
        _wait_w(_NCA - 3, 1)
        _compute(1)
        s1 = _scatter(_NCA - 1, 1)
        s0.wait()
        s1.wait()
        _wait_w(_NCA - 2, 0)
        _wait_w(_NCA - 1, 1)

    plsc.subcore_barrier()
    pltpu.sync_copy(sacc0.at[pl.ds(sid * _RPT, _RPT)],
                    s_out0.at[cid, pl.ds(sid * _RPT, _RPT)])
    pltpu.sync_copy(sacc1.at[pl.ds(sid * _RPT, _RPT)],
                    s_out1.at[cid, pl.ds(sid * _RPT, _RPT)])


def _attn(l0a, r0a, l1a, r1a, src0, dst0, src1, dst1):
    dbuf = lambda: [
        pltpu.VMEM((_CHA, 16), _F32),
        pltpu.VMEM((_CHA, 16), _F32),
        pltpu.VMEM((_CHA, 16), _F32),
    ]
    return pl.kernel(
        _attn_body,
        out_type=[
            jax.ShapeDtypeStruct((_NC, _NPAD, 16), _F32),
            jax.ShapeDtypeStruct((_NC, _NPAD, 16), _F32),
            jax.ShapeDtypeStruct((_EPAD, 16), _F32),
            jax.ShapeDtypeStruct((_EPAD, 16), _F32),
        ],
        mesh=_mesh(),
        compiler_params=_SC_PARAMS,
        scratch_types=[
            pltpu.VMEM((_NCA, _CHA), jnp.int32),
            pltpu.VMEM((_NCA, _CHA), jnp.int32),
        ] + dbuf() + dbuf() + [
            pltpu.VMEM((_RPT, 16), _F32),
            pltpu.VMEM_SHARED((_NPAD, 16), _F32),
            pltpu.VMEM_SHARED((_NPAD, 16), _F32),
            pltpu.SemaphoreType.DMA,
            pltpu.SemaphoreType.DMA,
            pltpu.SemaphoreType.DMA,
            pltpu.SemaphoreType.DMA,
            pltpu.SemaphoreType.DMA,
            pltpu.SemaphoreType.DMA,
        ],
    )(l0a, r0a, l1a, r1a, src0, dst0, src1, dst1)


def _ssum_body(a_ref, b_ref, oa_ref, ob_ref):
    oa_ref[...] = a_ref[0] + a_ref[1]
    ob_ref[...] = b_ref[0] + b_ref[1]


def _ssum(sA0, sA1):
    return pl.pallas_call(
        _ssum_body,
        grid=(_NBLK,),
        in_specs=[
            pl.BlockSpec((_NC, _BLK, 16), lambda i: (0, i, 0)),
            pl.BlockSpec((_NC, _BLK, 16), lambda i: (0, i, 0)),
        ],
        out_specs=[
            pl.BlockSpec((_BLK, 16), lambda i: (i, 0)),
            pl.BlockSpec((_BLK, 16), lambda i: (i, 0)),
        ],
        out_shape=[
            jax.ShapeDtypeStruct((_NPAD, 16), _F32),
            jax.ShapeDtypeStruct((_NPAD, 16), _F32),
        ],
    )(sA0, sA1)


def _spmm_body(f0, f1, st0, st1, src0, dst0, src1, dst1, exi0, exi1, out_hbm,
               isb, idb,
               ev0, sv0, fv0,
               ev1, sv1, fv1,
               zb2, oacc,
               semg0, semg1, semsc0, semsc1):
    cid = lax.axis_index("c")
    sid = lax.axis_index("s")
    wid = sid * _NC + cid
    SETS = ((ev0, sv0, fv0, semg0, semsc0),
            (ev1, sv1, fv1, semg1, semsc1))

    def _zrow(i, c):
        for j in range(8):
            zb2[i, pl.ds(16 * j, 16)] = jnp.zeros((16,), _F32)
        return c

    lax.fori_loop(0, 16, _zrow, 0)

    for p in range(_P):
        ft = (f0, f1)[p]
        st = (st0, st1)[p]
        se = (src0, src1)[p]
        de = (dst0, dst1)[p]
        ext = (exi0, exi1)[p]

        nz = _RPT // 16
        def _zacc(t, c):
            pltpu.async_copy(zb2, oacc.at[pl.ds(sid * _RPT + t * 16, 16)],
                             semg0)
            return c

        lax.fori_loop(0, nz, _zacc, 0)

        def _zwait(t, c):
            pltpu.make_async_copy(zb2, oacc.at[pl.ds(sid * _RPT, 16)],
                                  semg0).wait()
            return c

        lax.fori_loop(0, nz, _zwait, 0)
        plsc.subcore_barrier()

        pltpu.sync_copy(se.at[pl.ds(wid * _NCS, _NCS)], isb)
        pltpu.sync_copy(de.at[pl.ds(wid * _NCS, _NCS)], idb)

        def _off(k):
            return wid * _EPW + k * _CHS

        def _issue(k, b):
            ev, sv, fv, semg, semsc = SETS[b]
            pltpu.async_copy(ext.at[pl.ds(_off(k), _CHS)], ev, semg)
            pltpu.async_copy(st.at[idb.at[k]], sv, semg)
            pltpu.async_copy(ft.at[isb.at[k]], fv, semg)

        def _wait_g(k, b):
            ev, sv, fv, semg, semsc = SETS[b]
            pltpu.make_async_copy(ext.at[pl.ds(_off(k), _CHS)], ev, semg).wait()
            pltpu.make_async_copy(st.at[idb.at[k]], sv, semg).wait()
            pltpu.make_async_copy(ft.at[isb.at[k]], fv, semg).wait()

        def _compute(b):
            ev, sv, fv, semg, semsc = SETS[b]

            def _edge(i, cc):
                av = ev[i, :] / (sv[i, :] + 1e-9)
                for j in range(8):
                    fv[i, pl.ds(16 * j, 16)] = fv[i, pl.ds(16 * j, 16)] * av
                return cc

            lax.fori_loop(0, _CHS, _edge, 0, unroll=2)

        def _scatter(k, b):
            ev, sv, fv, semg, semsc = SETS[b]
            return pltpu.async_copy(fv, oacc.at[idb.at[k]], semsc, add=True)

        _issue(0, 0)
        _issue(1, 1)
        _wait_g(0, 0)
        _compute(0)
        c0 = _scatter(0, 0)
        _wait_g(1, 1)
        _compute(1)
        c1 = _scatter(1, 1)
        c0.wait()
        _issue(2, 0)
        c1.wait()
        _issue(3, 1)

        def _pipe(t, c):
            _wait_g(2 * t, 0)
            _compute(0)
            s0 = _scatter(2 * t, 0)
            _wait_g(2 * t + 1, 1)
            _compute(1)
            s1 = _scatter(2 * t + 1, 1)
            s0.wait()
            _issue(2 * t + 2, 0)
            s1.wait()
            _issue(2 * t + 3, 1)
            return c

        lax.fori_loop(1, _NCS // 2 - 1, _pipe, 0)
        _wait_g(_NCS - 2, 0)
        _compute(0)
        s0 = _scatter(_NCS - 2, 0)
        _wait_g(_NCS - 1, 1)
        _compute(1)
        s1 = _scatter(_NCS - 1, 1)
        s0.wait()
        s1.wait()

        plsc.subcore_barrier()
        pltpu.sync_copy(oacc.at[pl.ds(sid * _RPT, _RPT)],
                        out_hbm.at[p, cid, pl.ds(sid * _RPT, _RPT)])


def _spmm(f0a, f1a, st0, st1, src0, dst0, src1, dst1, exi0, exi1):
    dbuf = lambda: [
        pltpu.VMEM((_CHS, 16), _F32),
        pltpu.VMEM((_CHS, 16), _F32),
        pltpu.VMEM((_CHS, _HD), _F32),
    ]
    return pl.kernel(
        _spmm_body,
        out_type=jax.ShapeDtypeStruct((_P, _NC, _NPAD, _HD), _F32),
        mesh=_mesh(),
        compiler_params=_SC_PARAMS,
        scratch_types=[
            pltpu.VMEM((_NCS, _CHS), jnp.int32),
            pltpu.VMEM((_NCS, _CHS), jnp.int32),
        ] + dbuf() + dbuf() + [
            pltpu.VMEM((16, _HD), _F32),
            pltpu.VMEM_SHARED((_NPAD, _HD), _F32),
            pltpu.SemaphoreType.DMA,
            pltpu.SemaphoreType.DMA,
            pltpu.SemaphoreType.DMA,
            pltpu.SemaphoreType.DMA,
        ],
    )(f0a, f1a, st0, st1, src0, dst0, src1, dst1, exi0, exi1)


def _f1_body(op_ref, gt_ref, bg_ref, w1_ref, b1_ref, w2_ref, z_ref, ws_ref):
    nb = pl.program_id(1)
    operm = op_ref[0, 0] + op_ref[0, 1]
    o = jnp.dot(operm, gt_ref[...], preferred_element_type=_F32) + bg_ref[0, 0]
    z = jnp.where(o > 0, o, jnp.exp(o) - 1.0)
    z_ref[0] = z
    t = jnp.tanh(jnp.dot(z, w1_ref[...], preferred_element_type=_F32)
                 + b1_ref[...])
    wcol = jnp.sum(t * w2_ref[...], axis=1, keepdims=True)
    rows = nb * _BLK + lax.broadcasted_iota(jnp.int32, (_BLK, 1), 0)
    wcol = jnp.where(rows < _N, wcol, 0.0)
    sall = jnp.sum(wcol)

    @pl.when(nb == 0)
    def _():
        ws_ref[...] = jnp.full((1, 1, 128), sall, _F32)

    @pl.when(nb > 0)
    def _():
        ws_ref[...] = ws_ref[...] + sall


def _f1(outp, Gt, bias_gat, W_s1, b1r, w2r):
    return pl.pallas_call(
        _f1_body,
        grid=(_P, _NBLK),
        in_specs=[
            pl.BlockSpec((1, _NC, _BLK, _HD), lambda p, i: (p, 0, i, 0)),
            pl.BlockSpec((_HD, _HD), lambda p, i: (0, 0)),
            pl.BlockSpec((1, 1, _HD), lambda p, i: (p, 0, 0)),
            pl.BlockSpec((_HD, _HD), lambda p, i: (0, 0)),
            pl.BlockSpec((1, _HD), lambda p, i: (0, 0)),
            pl.BlockSpec((1, _HD), lambda p, i: (0, 0)),
        ],
        out_specs=[
            pl.BlockSpec((1, _BLK, _HD), lambda p, i: (p, i, 0)),
            pl.BlockSpec((1, 1, 128), lambda p, i: (p, 0, 0)),
        ],
        out_shape=[
            jax.ShapeDtypeStruct((_P, _NPAD, _HD), _F32),
            jax.ShapeDtypeStruct((_P, 1, 128), _F32),
        ],
    )(outp, Gt, bias_gat.reshape(_P, 1, _HD), W_s1, b1r, w2r)


def _f2_body(z_ref, ws_ref, wp_ref, bp_ref, o_ref):
    w = ws_ref[:, 0, :] / float(_N)
    m = jnp.max(w, axis=0, keepdims=True)
    ew = jnp.exp(w - m)
    beta = ew / jnp.sum(ew, axis=0, keepdims=True)
    h = z_ref[0] * beta[0:1, :] + z_ref[1] * beta[1:2, :]
    o_ref[...] = jnp.dot(h, wp_ref[...], preferred_element_type=_F32) + bp_ref[...]


def _f2(z, wsum, W_p, bpr):
    return pl.pallas_call(
        _f2_body,
        grid=(_NBLK,),
        in_specs=[
            pl.BlockSpec((_P, _BLK, _HD), lambda i: (0, i, 0)),
            pl.BlockSpec((_P, 1, 128), lambda i: (0, 0, 0)),
            pl.BlockSpec((_HD, _C), lambda i: (0, 0)),
            pl.BlockSpec((1, _C), lambda i: (0, 0)),
        ],
        out_specs=pl.BlockSpec((_BLK, _C), lambda i: (i, 0)),
        out_shape=jax.ShapeDtypeStruct((_NPAD, _C), _F32),
    )(z, wsum, W_p, bpr)


def kernel(x, W_gat, attn_l, attn_r, bias_gat, W_s1, b_s1, W_s2, W_p, b_p,
           edge_index_0, edge_index_1):
    xpad = jnp.zeros((_NPAD, _FIN), _F32).at[:_N].set(x)
    rows = jnp.arange(_HD)
    hcol = rows // _D
    AL = jnp.zeros((_P, _HD, 16), _F32)
    AL = AL.at[:, rows, hcol].set(attn_l.reshape(_P, _HD))
    AL = AL.at[:, rows, hcol + 8].set(attn_l.reshape(_P, _HD))
    AR = jnp.zeros((_P, _HD, 16), _F32)
    AR = AR.at[:, rows, hcol].set(attn_r.reshape(_P, _HD))
    AR = AR.at[:, rows, hcol + 8].set(attn_r.reshape(_P, _HD))
    cc = jnp.arange(_HD)
    ll = cc % 16
    kk = cc // 16
    gidx = (ll % 8) * _D + 2 * kk + ll // 8
    G = jnp.zeros((_HD, _HD), _F32).at[gidx, cc].set(1.0)
    Gt = G.T

    feat, L, R = _prep(xpad, W_gat, AL, AR, G)
    pad = _N + (jnp.arange(_EPAD - _E, dtype=jnp.int32) % (_NPAD - _N))
    s0f = jnp.concatenate([edge_index_0[0], pad])
    d0f = jnp.concatenate([edge_index_0[1], pad])
    s1f = jnp.concatenate([edge_index_1[0], pad])
    d1f = jnp.concatenate([edge_index_1[1], pad])

    ra = lambda a: a.reshape(_EPAD // _CHA, _CHA)
    rs = lambda a: a.reshape(_EPAD // _CHS, _CHS)

    sA0, sA1, exA0, exA1 = _attn(L[0], R[0], L[1], R[1],
                                 ra(s0f), ra(d0f), ra(s1f), ra(d1f))
    st0, st1 = _ssum(sA0, sA1)
    outp = _spmm(feat[0], feat[1], st0, st1,
                 rs(s0f), rs(d0f), rs(s1f), rs(d1f),
                 exA0, exA1)
    z, wsum = _f1(outp, Gt, bias_gat, W_s1, b_s1.reshape(1, -1),
                  W_s2.reshape(1, -1))
    out = _f2(z, wsum, W_p, b_p.reshape(1, -1))
    return out[:_N]


# speedup vs baseline: 1.1039x; 1.1039x over previous
import jax
import jax.numpy as jnp
from jax import lax
from jax.experimental import pallas as pl
from jax.experimental.pallas import tpu as pltpu
from jax.experimental.pallas import tpu_sc as plsc

_N = 10000
_E = 320000
_FIN = 128
_H = 8
_D = 16
_HD = 128
_P = 2
_C = 16

_NPAD = 10240
_NC = 2
_NS = 16
_NW = _NC * _NS
_EPW = 10240
_EPAD = _EPW * _NW
_CHA = 128
_NCA = _EPW // _CHA
_CHS = 80
_NCS = _EPW // _CHS
_RPT = _NPAD // _NS
_BLK = 1024
_NBLK = _NPAD // _BLK

_F32 = jnp.float32
_SC_PARAMS = pltpu.CompilerParams(use_tc_tiling_on_sc=False)


def _mesh():
    return plsc.VectorSubcoreMesh(core_axis_name="c", subcore_axis_name="s",
                                  num_cores=_NC, num_subcores=_NS)


def _prep_body(x_ref, w_ref, al_ref, ar_ref, g_ref, feat_ref, l_ref, r_ref):
    f = jnp.dot(x_ref[...], w_ref[0], preferred_element_type=_F32)
    feat_ref[0] = jnp.dot(f, g_ref[...], preferred_element_type=_F32)
    l_ref[0] = jnp.dot(f, al_ref[0], preferred_element_type=_F32)
    r_ref[0] = jnp.dot(f, ar_ref[0], preferred_element_type=_F32)


def _prep(xpad, W_gat, AL, AR, G):
    return pl.pallas_call(
        _prep_body,
        grid=(_P, _NBLK),
        in_specs=[
            pl.BlockSpec((_BLK, _FIN), lambda p, i: (i, 0)),
            pl.BlockSpec((1, _FIN, _HD), lambda p, i: (p, 0, 0)),
            pl.BlockSpec((1, _HD, 16), lambda p, i: (p, 0, 0)),
            pl.BlockSpec((1, _HD, 16), lambda p, i: (p, 0, 0)),
            pl.BlockSpec((_HD, _HD), lambda p, i: (0, 0)),
        ],
        out_specs=[
            pl.BlockSpec((1, _BLK, _HD), lambda p, i: (p, i, 0)),
            pl.BlockSpec((1, _BLK, 16), lambda p, i: (p, i, 0)),
            pl.BlockSpec((1, _BLK, 16), lambda p, i: (p, i, 0)),
        ],
        out_shape=[
            jax.ShapeDtypeStruct((_P, _NPAD, _HD), _F32),
            jax.ShapeDtypeStruct((_P, _NPAD, 16), _F32),
            jax.ShapeDtypeStruct((_P, _NPAD, 16), _F32),
        ],
    )(xpad, W_gat, AL, AR, G)


def _attn_body(l0, r0, l1, r1, src0, dst0, src1, dst1, s_out0, s_out1,
               ex0, ex1,
               isb, idb,
               lv0, rv0, ev0,
               lv1, rv1, ev1,
               zb, sacc0, sacc1,
               semg0, semg1, semw0, semw1, semsc0, semsc1):
    cid = lax.axis_index("c")
    sid = lax.axis_index("s")
    wid = sid * _NC + cid
    SETS = ((lv0, rv0, ev0, semg0, semw0, semsc0),
            (lv1, rv1, ev1, semg1, semw1, semsc1))

    def _zrow(i, c):
        zb[i, :] = jnp.zeros((16,), _F32)
        return c

    lax.fori_loop(0, _RPT, _zrow, 0)
    pltpu.sync_copy(zb, sacc0.at[pl.ds(sid * _RPT, _RPT)])
    pltpu.sync_copy(zb, sacc1.at[pl.ds(sid * _RPT, _RPT)])
    plsc.subcore_barrier()

    for p in range(_P):
        lt = (l0, l1)[p]
        rt = (r0, r1)[p]
        se = (src0, src1)[p]
        de = (dst0, dst1)[p]
        sacc = (sacc0, sacc1)[p]
        ext = (ex0, ex1)[p]

        pltpu.sync_copy(se.at[pl.ds(wid * _NCA, _NCA)], isb)
        pltpu.sync_copy(de.at[pl.ds(wid * _NCA, _NCA)], idb)

        def _off(k):
            return wid * _EPW + k * _CHA

        def _issue(k, b):
            lv, rv, ev, semg, semw, semsc = SETS[b]
            pltpu.async_copy(lt.at[isb.at[k]], lv, semg)
            pltpu.async_copy(rt.at[idb.at[k]], rv, semg)

        def _wait_g(k, b):
            lv, rv, ev, semg, semw, semsc = SETS[b]
            pltpu.make_async_copy(lt.at[isb.at[k]], lv, semg).wait()
            pltpu.make_async_copy(rt.at[idb.at[k]], rv, semg).wait()

        def _compute(b):
            lv, rv, ev, semg, semw, semsc = SETS[b]

            def _edge(i, cc):
                e = lv[i, :] + rv[i, :]
                e = jnp.where(e > 0, e, 0.2 * e)
                ev[i, :] = jnp.exp(e)
                return cc

            lax.fori_loop(0, _CHA, _edge, 0, unroll=4)

        def _scatter(k, b):
            lv, rv, ev, semg, semw, semsc = SETS[b]
            pltpu.async_copy(ev, ext.at[pl.ds(_off(k), _CHA)], semw)
            pltpu.sync_copy(ev, sacc.at[idb.at[k]], add=True)

        def _wait_w(k, b):
            lv, rv, ev, semg, semw, semsc = SETS[b]
            pltpu.make_async_copy(ev, ext.at[pl.ds(_off(k), _CHA)], semw).wait()

        _issue(0, 0)
        _issue(1, 1)
        _wait_g(0, 0)
        _compute(0)
        _scatter(0, 0)
        _issue(2, 0)
        _wait_g(1, 1)
        _compute(1)
        _scatter(1, 1)
        _issue(3, 1)

        def _pipe(t, c):
            _wait_g(2 * t, 0)
            _wait_w(2 * t - 2, 0)
            _compute(0)
            _scatter(2 * t, 0)
            _issue(2 * t + 2, 0)
            _wait_g(2 * t + 1, 1)
            _wait_w(2 * t - 1, 1)
            _compute(1)
            _scatter(2 * t + 1, 1)
            _issue(2 * t + 3, 1)
            return c

        lax.fori_loop(1, _NCA // 2 - 1, _pipe, 0)
        _wait_g(_NCA - 2, 0)
        _wait_w(_NCA - 4, 0)
        _compute(0)
        _scatter(_NCA - 2, 0)
        _wait_g(_NCA - 1, 1)
        _wait_w(_NCA - 3, 1)
        _compute(1)
        _scatter(_NCA - 1, 1)
        _wait_w(_NCA - 2, 0)
        _wait_w(_NCA - 1, 1)

    plsc.subcore_barrier()
    pltpu.sync_copy(sacc0.at[pl.ds(sid * _RPT, _RPT)],
                    s_out0.at[cid, pl.ds(sid * _RPT, _RPT)])
    pltpu.sync_copy(sacc1.at[pl.ds(sid * _RPT, _RPT)],
                    s_out1.at[cid, pl.ds(sid * _RPT, _RPT)])


def _attn(l0a, r0a, l1a, r1a, src0, dst0, src1, dst1):
    dbuf = lambda: [
        pltpu.VMEM((_CHA, 16), _F32),
        pltpu.VMEM((_CHA, 16), _F32),
        pltpu.VMEM((_CHA, 16), _F32),
    ]
    return pl.kernel(
        _attn_body,
        out_type=[
            jax.ShapeDtypeStruct((_NC, _NPAD, 16), _F32),
            jax.ShapeDtypeStruct((_NC, _NPAD, 16), _F32),
            jax.ShapeDtypeStruct((_EPAD, 16), _F32),
            jax.ShapeDtypeStruct((_EPAD, 16), _F32),
        ],
        mesh=_mesh(),
        compiler_params=_SC_PARAMS,
        scratch_types=[
            pltpu.VMEM((_NCA, _CHA), jnp.int32),
            pltpu.VMEM((_NCA, _CHA), jnp.int32),
        ] + dbuf() + dbuf() + [
            pltpu.VMEM((_RPT, 16), _F32),
            pltpu.VMEM_SHARED((_NPAD, 16), _F32),
            pltpu.VMEM_SHARED((_NPAD, 16), _F32),
            pltpu.SemaphoreType.DMA,
            pltpu.SemaphoreType.DMA,
            pltpu.SemaphoreType.DMA,
            pltpu.SemaphoreType.DMA,
            pltpu.SemaphoreType.DMA,
            pltpu.SemaphoreType.DMA,
        ],
    )(l0a, r0a, l1a, r1a, src0, dst0, src1, dst1)


def _ssum_body(a_ref, b_ref, oa_ref, ob_ref):
    oa_ref[...] = a_ref[0] + a_ref[1]
    ob_ref[...] = b_ref[0] + b_ref[1]


def _ssum(sA0, sA1):
    return pl.pallas_call(
        _ssum_body,
        grid=(_NBLK,),
        in_specs=[
            pl.BlockSpec((_NC, _BLK, 16), lambda i: (0, i, 0)),
            pl.BlockSpec((_NC, _BLK, 16), lambda i: (0, i, 0)),
        ],
        out_specs=[
            pl.BlockSpec((_BLK, 16), lambda i: (i, 0)),
            pl.BlockSpec((_BLK, 16), lambda i: (i, 0)),
        ],
        out_shape=[
            jax.ShapeDtypeStruct((_NPAD, 16), _F32),
            jax.ShapeDtypeStruct((_NPAD, 16), _F32),
        ],
    )(sA0, sA1)


def _spmm_body(f0, f1, st0, st1, src0, dst0, src1, dst1, exi0, exi1, out_hbm,
               isb, idb,
               ev0, sv0, fv0,
               ev1, sv1, fv1,
               zb2, oacc,
               semg0, semg1, semsc0, semsc1):
    cid = lax.axis_index("c")
    sid = lax.axis_index("s")
    wid = sid * _NC + cid
    SETS = ((ev0, sv0, fv0, semg0, semsc0),
            (ev1, sv1, fv1, semg1, semsc1))

    def _zrow(i, c):
        for j in range(8):
            zb2[i, pl.ds(16 * j, 16)] = jnp.zeros((16,), _F32)
        return c

    lax.fori_loop(0, 16, _zrow, 0)

    for p in range(_P):
        ft = (f0, f1)[p]
        st = (st0, st1)[p]
        se = (src0, src1)[p]
        de = (dst0, dst1)[p]
        ext = (exi0, exi1)[p]

        nz = _RPT // 16
        def _zacc(t, c):
            pltpu.async_copy(zb2, oacc.at[pl.ds(sid * _RPT + t * 16, 16)],
                             semg0)
            return c

        lax.fori_loop(0, nz, _zacc, 0)

        def _zwait(t, c):
            pltpu.make_async_copy(zb2, oacc.at[pl.ds(sid * _RPT, 16)],
                                  semg0).wait()
            return c

        lax.fori_loop(0, nz, _zwait, 0)
        plsc.subcore_barrier()

        pltpu.sync_copy(se.at[pl.ds(wid * _NCS, _NCS)], isb)
        pltpu.sync_copy(de.at[pl.ds(wid * _NCS, _NCS)], idb)

        def _off(k):
            return wid * _EPW + k * _CHS

        def _issue(k, b):
            ev, sv, fv, semg, semsc = SETS[b]
            pltpu.async_copy(ext.at[pl.ds(_off(k), _CHS)], ev, semg)
            pltpu.async_copy(st.at[idb.at[k]], sv, semg)
            pltpu.async_copy(ft.at[isb.at[k]], fv, semg)

        def _wait_g(k, b):
            ev, sv, fv, semg, semsc = SETS[b]
            pltpu.make_async_copy(ext.at[pl.ds(_off(k), _CHS)], ev, semg).wait()
            pltpu.make_async_copy(st.at[idb.at[k]], sv, semg).wait()
            pltpu.make_async_copy(ft.at[isb.at[k]], fv, semg).wait()

        def _compute(b):
            ev, sv, fv, semg, semsc = SETS[b]

            def _edge(i, cc):
                av = ev[i, :] / (sv[i, :] + 1e-9)
                for j in range(8):
                    fv[i, pl.ds(16 * j, 16)] = fv[i, pl.ds(16 * j, 16)] * av
                return cc

            lax.fori_loop(0, _CHS, _edge, 0, unroll=2)

        def _scatter(k, b):
            ev, sv, fv, semg, semsc = SETS[b]
            pltpu.sync_copy(fv, oacc.at[idb.at[k]], add=True)

        _issue(0, 0)
        _issue(1, 1)
        _wait_g(0, 0)
        _compute(0)
        _scatter(0, 0)
        _issue(2, 0)
        _wait_g(1, 1)
        _compute(1)
        _scatter(1, 1)
        _issue(3, 1)

        def _pipe(t, c):
            _wait_g(2 * t, 0)
            _compute(0)
            _scatter(2 * t, 0)
            _issue(2 * t + 2, 0)
            _wait_g(2 * t + 1, 1)
            _compute(1)
            _scatter(2 * t + 1, 1)
            _issue(2 * t + 3, 1)
            return c

        lax.fori_loop(1, _NCS // 2 - 1, _pipe, 0)
        _wait_g(_NCS - 2, 0)
        _compute(0)
        _scatter(_NCS - 2, 0)
        _wait_g(_NCS - 1, 1)
        _compute(1)
        _scatter(_NCS - 1, 1)

        plsc.subcore_barrier()
        pltpu.sync_copy(oacc.at[pl.ds(sid * _RPT, _RPT)],
                        out_hbm.at[p, cid, pl.ds(sid * _RPT, _RPT)])


def _spmm(f0a, f1a, st0, st1, src0, dst0, src1, dst1, exi0, exi1):
    dbuf = lambda: [
        pltpu.VMEM((_CHS, 16), _F32),
        pltpu.VMEM((_CHS, 16), _F32),
        pltpu.VMEM((_CHS, _HD), _F32),
    ]
    return pl.kernel(
        _spmm_body,
        out_type=jax.ShapeDtypeStruct((_P, _NC, _NPAD, _HD), _F32),
        mesh=_mesh(),
        compiler_params=_SC_PARAMS,
        scratch_types=[
            pltpu.VMEM((_NCS, _CHS), jnp.int32),
            pltpu.VMEM((_NCS, _CHS), jnp.int32),
        ] + dbuf() + dbuf() + [
            pltpu.VMEM((16, _HD), _F32),
            pltpu.VMEM_SHARED((_NPAD, _HD), _F32),
            pltpu.SemaphoreType.DMA,
            pltpu.SemaphoreType.DMA,
            pltpu.SemaphoreType.DMA,
            pltpu.SemaphoreType.DMA,
        ],
    )(f0a, f1a, st0, st1, src0, dst0, src1, dst1, exi0, exi1)


def _f1_body(op_ref, gt_ref, bg_ref, w1_ref, b1_ref, w2_ref, z_ref, ws_ref):
    nb = pl.program_id(1)
    operm = op_ref[0, 0] + op_ref[0, 1]
    o = jnp.dot(operm, gt_ref[...], preferred_element_type=_F32) + bg_ref[0, 0]
    z = jnp.where(o > 0, o, jnp.exp(o) - 1.0)
    z_ref[0] = z
    t = jnp.tanh(jnp.dot(z, w1_ref[...], preferred_element_type=_F32)
                 + b1_ref[...])
    wcol = jnp.sum(t * w2_ref[...], axis=1, keepdims=True)
    rows = nb * _BLK + lax.broadcasted_iota(jnp.int32, (_BLK, 1), 0)
    wcol = jnp.where(rows < _N, wcol, 0.0)
    sall = jnp.sum(wcol)

    @pl.when(nb == 0)
    def _():
        ws_ref[...] = jnp.full((1, 1, 128), sall, _F32)

    @pl.when(nb > 0)
    def _():
        ws_ref[...] = ws_ref[...] + sall


def _f1(outp, Gt, bias_gat, W_s1, b1r, w2r):
    return pl.pallas_call(
        _f1_body,
        grid=(_P, _NBLK),
        in_specs=[
            pl.BlockSpec((1, _NC, _BLK, _HD), lambda p, i: (p, 0, i, 0)),
            pl.BlockSpec((_HD, _HD), lambda p, i: (0, 0)),
            pl.BlockSpec((1, 1, _HD), lambda p, i: (p, 0, 0)),
            pl.BlockSpec((_HD, _HD), lambda p, i: (0, 0)),
            pl.BlockSpec((1, _HD), lambda p, i: (0, 0)),
            pl.BlockSpec((1, _HD), lambda p, i: (0, 0)),
        ],
        out_specs=[
            pl.BlockSpec((1, _BLK, _HD), lambda p, i: (p, i, 0)),
            pl.BlockSpec((1, 1, 128), lambda p, i: (p, 0, 0)),
        ],
        out_shape=[
            jax.ShapeDtypeStruct((_P, _NPAD, _HD), _F32),
            jax.ShapeDtypeStruct((_P, 1, 128), _F32),
        ],
    )(outp, Gt, bias_gat.reshape(_P, 1, _HD), W_s1, b1r, w2r)


def _f2_body(z_ref, ws_ref, wp_ref, bp_ref, o_ref):
    w = ws_ref[:, 0, :] / float(_N)
    m = jnp.max(w, axis=0, keepdims=True)
    ew = jnp.exp(w - m)
    beta = ew / jnp.sum(ew, axis=0, keepdims=True)
    h = z_ref[0] * beta[0:1, :] + z_ref[1] * beta[1:2, :]
    o_ref[...] = jnp.dot(h, wp_ref[...], preferred_element_type=_F32) + bp_ref[...]


def _f2(z, wsum, W_p, bpr):
    return pl.pallas_call(
        _f2_body,
        grid=(_NBLK,),
        in_specs=[
            pl.BlockSpec((_P, _BLK, _HD), lambda i: (0, i, 0)),
            pl.BlockSpec((_P, 1, 128), lambda i: (0, 0, 0)),
            pl.BlockSpec((_HD, _C), lambda i: (0, 0)),
            pl.BlockSpec((1, _C), lambda i: (0, 0)),
        ],
        out_specs=pl.BlockSpec((_BLK, _C), lambda i: (i, 0)),
        out_shape=jax.ShapeDtypeStruct((_NPAD, _C), _F32),
    )(z, wsum, W_p, bpr)


def kernel(x, W_gat, attn_l, attn_r, bias_gat, W_s1, b_s1, W_s2, W_p, b_p,
           edge_index_0, edge_index_1):
    xpad = jnp.zeros((_NPAD, _FIN), _F32).at[:_N].set(x)
    rows = jnp.arange(_HD)
    hcol = rows // _D
    AL = jnp.zeros((_P, _HD, 16), _F32)
    AL = AL.at[:, rows, hcol].set(attn_l.reshape(_P, _HD))
    AL = AL.at[:, rows, hcol + 8].set(attn_l.reshape(_P, _HD))
    AR = jnp.zeros((_P, _HD, 16), _F32)
    AR = AR.at[:, rows, hcol].set(attn_r.reshape(_P, _HD))
    AR = AR.at[:, rows, hcol + 8].set(attn_r.reshape(_P, _HD))
    cc = jnp.arange(_HD)
    ll = cc % 16
    kk = cc // 16
    gidx = (ll % 8) * _D + 2 * kk + ll // 8
    G = jnp.zeros((_HD, _HD), _F32).at[gidx, cc].set(1.0)
    Gt = G.T

    feat, L, R = _prep(xpad, W_gat, AL, AR, G)
    pad = _N + (jnp.arange(_EPAD - _E, dtype=jnp.int32) % (_NPAD - _N))
    s0f = jnp.concatenate([edge_index_0[0], pad])
    d0f = jnp.concatenate([edge_index_0[1], pad])
    s1f = jnp.concatenate([edge_index_1[0], pad])
    d1f = jnp.concatenate([edge_index_1[1], pad])

    ra = lambda a: a.reshape(_EPAD // _CHA, _CHA)
    rs = lambda a: a.reshape(_EPAD // _CHS, _CHS)

    sA0, sA1, exA0, exA1 = _attn(L[0], R[0], L[1], R[1],
                                 ra(s0f), ra(d0f), ra(s1f), ra(d1f))
    st0, st1 = _ssum(sA0, sA1)
    outp = _spmm(feat[0], feat[1], st0, st1,
                 rs(s0f), rs(d0f), rs(s1f), rs(d1f),
                 exA0, exA1)
    z, wsum = _f1(outp, Gt, bias_gat, W_s1, b_s1.reshape(1, -1),
                  W_s2.reshape(1, -1))
    out = _f2(z, wsum, W_p, b_p.reshape(1, -1))
    return out[:_N]


# speedup vs baseline: 1.2529x; 1.1350x over previous
import jax
import jax.numpy as jnp
from jax import lax
from jax.experimental import pallas as pl
from jax.experimental.pallas import tpu as pltpu
from jax.experimental.pallas import tpu_sc as plsc

_N = 10000
_E = 320000
_FIN = 128
_H = 8
_D = 16
_HD = 128
_P = 2
_C = 16

_NPAD = 10240
_NC = 2
_NS = 16
_NW = _NC * _NS
_EPW = 10240
_EPAD = _EPW * _NW
_CHA = 128
_NCA = _EPW // _CHA
_CHS = 80
_NCS = _EPW // _CHS
_RPT = _NPAD // _NS
_BLK = 1024
_NBLK = _NPAD // _BLK

_F32 = jnp.float32
_SC_PARAMS = pltpu.CompilerParams(use_tc_tiling_on_sc=False)


def _mesh():
    return plsc.VectorSubcoreMesh(core_axis_name="c", subcore_axis_name="s",
                                  num_cores=_NC, num_subcores=_NS)


def _prep_body(x_ref, w_ref, al_ref, ar_ref, g_ref, feat_ref, l_ref, r_ref):
    f = jnp.dot(x_ref[...], w_ref[0], preferred_element_type=_F32)
    feat_ref[0] = jnp.dot(f, g_ref[...], preferred_element_type=_F32)
    l_ref[0] = jnp.dot(f, al_ref[0], preferred_element_type=_F32)
    r_ref[0] = jnp.dot(f, ar_ref[0], preferred_element_type=_F32)


def _prep(xpad, W_gat, AL, AR, G):
    return pl.pallas_call(
        _prep_body,
        grid=(_P, _NBLK),
        in_specs=[
            pl.BlockSpec((_BLK, _FIN), lambda p, i: (i, 0)),
            pl.BlockSpec((1, _FIN, _HD), lambda p, i: (p, 0, 0)),
            pl.BlockSpec((1, _HD, 16), lambda p, i: (p, 0, 0)),
            pl.BlockSpec((1, _HD, 16), lambda p, i: (p, 0, 0)),
            pl.BlockSpec((_HD, _HD), lambda p, i: (0, 0)),
        ],
        out_specs=[
            pl.BlockSpec((1, _BLK, _HD), lambda p, i: (p, i, 0)),
            pl.BlockSpec((1, _BLK, 16), lambda p, i: (p, i, 0)),
            pl.BlockSpec((1, _BLK, 16), lambda p, i: (p, i, 0)),
        ],
        out_shape=[
            jax.ShapeDtypeStruct((_P, _NPAD, _HD), _F32),
            jax.ShapeDtypeStruct((_P, _NPAD, 16), _F32),
            jax.ShapeDtypeStruct((_P, _NPAD, 16), _F32),
        ],
    )(xpad, W_gat, AL, AR, G)


def _attn_body(l0, r0, l1, r1, src0, dst0, src1, dst1, s_out0, s_out1,
               ex0, ex1,
               isb, idb,
               lv0, rv0, ev0,
               lv1, rv1, ev1,
               zb, sacc0, sacc1,
               semg0, semg1, semw0, semw1, semsc0, semsc1):
    cid = lax.axis_index("c")
    sid = lax.axis_index("s")
    wid = sid * _NC + cid
    SETS = ((lv0, rv0, ev0, semg0, semw0, semsc0),
            (lv1, rv1, ev1, semg1, semw1, semsc1))

    def _zrow(i, c):
        zb[i, :] = jnp.zeros((16,), _F32)
        return c

    lax.fori_loop(0, _RPT, _zrow, 0)
    pltpu.sync_copy(zb, sacc0.at[pl.ds(sid * _RPT, _RPT)])
    pltpu.sync_copy(zb, sacc1.at[pl.ds(sid * _RPT, _RPT)])
    plsc.subcore_barrier()

    for p in range(_P):
        lt = (l0, l1)[p]
        rt = (r0, r1)[p]
        se = (src0, src1)[p]
        de = (dst0, dst1)[p]
        sacc = (sacc0, sacc1)[p]
        ext = (ex0, ex1)[p]

        pltpu.sync_copy(se.at[pl.ds(wid * _NCA, _NCA)], isb)
        pltpu.sync_copy(de.at[pl.ds(wid * _NCA, _NCA)], idb)

        def _off(k):
            return wid * _EPW + k * _CHA

        def _issue(k, b):
            lv, rv, ev, semg, semw, semsc = SETS[b]
            pltpu.async_copy(lt.at[isb.at[k]], lv, semg)
            pltpu.async_copy(rt.at[idb.at[k]], rv, semg)

        def _wait_g(k, b):
            lv, rv, ev, semg, semw, semsc = SETS[b]
            pltpu.make_async_copy(lt.at[isb.at[k]], lv, semg).wait()
            pltpu.make_async_copy(rt.at[idb.at[k]], rv, semg).wait()

        def _compute(b):
            lv, rv, ev, semg, semw, semsc = SETS[b]

            def _edge(i, cc):
                e = lv[i, :] + rv[i, :]
                e = jnp.where(e > 0, e, 0.2 * e)
                ev[i, :] = jnp.exp(e)
                return cc

            lax.fori_loop(0, _CHA, _edge, 0, unroll=4)

        def _scatter(k, b):
            lv, rv, ev, semg, semw, semsc = SETS[b]
            pltpu.async_copy(ev, ext.at[pl.ds(_off(k), _CHA)], semw)
            pltpu.sync_copy(ev, sacc.at[idb.at[k]], add=True)

        def _wait_w(k, b):
            lv, rv, ev, semg, semw, semsc = SETS[b]
            pltpu.make_async_copy(ev, ext.at[pl.ds(_off(k), _CHA)], semw).wait()

        _issue(0, 0)
        _issue(1, 1)
        _wait_g(0, 0)
        _compute(0)
        _scatter(0, 0)
        _issue(2, 0)
        _wait_g(1, 1)
        _compute(1)
        _scatter(1, 1)
        _issue(3, 1)

        def _pipe(t, c):
            _wait_g(2 * t, 0)
            _wait_w(2 * t - 2, 0)
            _compute(0)
            _scatter(2 * t, 0)
            _issue(2 * t + 2, 0)
            _wait_g(2 * t + 1, 1)
            _wait_w(2 * t - 1, 1)
            _compute(1)
            _scatter(2 * t + 1, 1)
            _issue(2 * t + 3, 1)
            return c

        lax.fori_loop(1, _NCA // 2 - 1, _pipe, 0)
        _wait_g(_NCA - 2, 0)
        _wait_w(_NCA - 4, 0)
        _compute(0)
        _scatter(_NCA - 2, 0)
        _wait_g(_NCA - 1, 1)
        _wait_w(_NCA - 3, 1)
        _compute(1)
        _scatter(_NCA - 1, 1)
        _wait_w(_NCA - 2, 0)
        _wait_w(_NCA - 1, 1)

    plsc.subcore_barrier()
    pltpu.sync_copy(sacc0.at[pl.ds(sid * _RPT, _RPT)],
                    s_out0.at[cid, pl.ds(sid * _RPT, _RPT)])
    pltpu.sync_copy(sacc1.at[pl.ds(sid * _RPT, _RPT)],
                    s_out1.at[cid, pl.ds(sid * _RPT, _RPT)])


def _attn(l0a, r0a, l1a, r1a, src0, dst0, src1, dst1):
    dbuf = lambda: [
        pltpu.VMEM((_CHA, 16), _F32),
        pltpu.VMEM((_CHA, 16), _F32),
        pltpu.VMEM((_CHA, 16), _F32),
    ]
    return pl.kernel(
        _attn_body,
        out_type=[
            jax.ShapeDtypeStruct((_NC, _NPAD, 16), _F32),
            jax.ShapeDtypeStruct((_NC, _NPAD, 16), _F32),
            jax.ShapeDtypeStruct((_EPAD, 16), _F32),
            jax.ShapeDtypeStruct((_EPAD, 16), _F32),
        ],
        mesh=_mesh(),
        compiler_params=_SC_PARAMS,
        scratch_types=[
            pltpu.VMEM((_NCA, _CHA), jnp.int32),
            pltpu.VMEM((_NCA, _CHA), jnp.int32),
        ] + dbuf() + dbuf() + [
            pltpu.VMEM((_RPT, 16), _F32),
            pltpu.VMEM_SHARED((_NPAD, 16), _F32),
            pltpu.VMEM_SHARED((_NPAD, 16), _F32),
            pltpu.SemaphoreType.DMA,
            pltpu.SemaphoreType.DMA,
            pltpu.SemaphoreType.DMA,
            pltpu.SemaphoreType.DMA,
            pltpu.SemaphoreType.DMA,
            pltpu.SemaphoreType.DMA,
        ],
    )(l0a, r0a, l1a, r1a, src0, dst0, src1, dst1)


def _ssum_body(a_ref, b_ref, oa_ref, ob_ref):
    oa_ref[...] = 1.0 / (a_ref[0] + a_ref[1] + 1e-9)
    ob_ref[...] = 1.0 / (b_ref[0] + b_ref[1] + 1e-9)


def _ssum(sA0, sA1):
    return pl.pallas_call(
        _ssum_body,
        grid=(_NBLK,),
        in_specs=[
            pl.BlockSpec((_NC, _BLK, 16), lambda i: (0, i, 0)),
            pl.BlockSpec((_NC, _BLK, 16), lambda i: (0, i, 0)),
        ],
        out_specs=[
            pl.BlockSpec((_BLK, 16), lambda i: (i, 0)),
            pl.BlockSpec((_BLK, 16), lambda i: (i, 0)),
        ],
        out_shape=[
            jax.ShapeDtypeStruct((_NPAD, 16), _F32),
            jax.ShapeDtypeStruct((_NPAD, 16), _F32),
        ],
    )(sA0, sA1)


def _spmm_body(f0, f1, st0, st1, src0, dst0, src1, dst1, exi0, exi1, out_hbm,
               isb, idb,
               ev0, sv0, fv0,
               ev1, sv1, fv1,
               zb2, oacc,
               semg0, semg1, semsc0, semsc1):
    cid = lax.axis_index("c")
    sid = lax.axis_index("s")
    wid = sid * _NC + cid
    SETS = ((ev0, sv0, fv0, semg0, semsc0),
            (ev1, sv1, fv1, semg1, semsc1))

    def _zrow(i, c):
        for j in range(8):
            zb2[i, pl.ds(16 * j, 16)] = jnp.zeros((16,), _F32)
        return c

    lax.fori_loop(0, 16, _zrow, 0)

    for p in range(_P):
        ft = (f0, f1)[p]
        st = (st0, st1)[p]
        se = (src0, src1)[p]
        de = (dst0, dst1)[p]
        ext = (exi0, exi1)[p]

        nz = _RPT // 16
        def _zacc(t, c):
            pltpu.async_copy(zb2, oacc.at[pl.ds(sid * _RPT + t * 16, 16)],
                             semg0)
            return c

        lax.fori_loop(0, nz, _zacc, 0)

        def _zwait(t, c):
            pltpu.make_async_copy(zb2, oacc.at[pl.ds(sid * _RPT, 16)],
                                  semg0).wait()
            return c

        lax.fori_loop(0, nz, _zwait, 0)
        plsc.subcore_barrier()

        pltpu.sync_copy(se.at[pl.ds(wid * _NCS, _NCS)], isb)
        pltpu.sync_copy(de.at[pl.ds(wid * _NCS, _NCS)], idb)

        def _off(k):
            return wid * _EPW + k * _CHS

        def _issue(k, b):
            ev, sv, fv, semg, semsc = SETS[b]
            pltpu.async_copy(ext.at[pl.ds(_off(k), _CHS)], ev, semg)
            pltpu.async_copy(st.at[idb.at[k]], sv, semg)
            pltpu.async_copy(ft.at[isb.at[k]], fv, semg)

        def _wait_g(k, b):
            ev, sv, fv, semg, semsc = SETS[b]
            pltpu.make_async_copy(ext.at[pl.ds(_off(k), _CHS)], ev, semg).wait()
            pltpu.make_async_copy(st.at[idb.at[k]], sv, semg).wait()
            pltpu.make_async_copy(ft.at[isb.at[k]], fv, semg).wait()

        def _compute(b):
            ev, sv, fv, semg, semsc = SETS[b]

            def _edge(i, cc):
                av = ev[i, :] * sv[i, :]
                for j in range(8):
                    fv[i, pl.ds(16 * j, 16)] = fv[i, pl.ds(16 * j, 16)] * av
                return cc

            lax.fori_loop(0, _CHS, _edge, 0, unroll=4)

        def _scatter(k, b):
            ev, sv, fv, semg, semsc = SETS[b]
            pltpu.sync_copy(fv, oacc.at[idb.at[k]], add=True)

        _issue(0, 0)
        _issue(1, 1)
        _wait_g(0, 0)
        _compute(0)
        _scatter(0, 0)
        _issue(2, 0)
        _wait_g(1, 1)
        _compute(1)
        _scatter(1, 1)
        _issue(3, 1)

        def _pipe(t, c):
            _wait_g(2 * t, 0)
            _compute(0)
            _scatter(2 * t, 0)
            _issue(2 * t + 2, 0)
            _wait_g(2 * t + 1, 1)
            _compute(1)
            _scatter(2 * t + 1, 1)
            _issue(2 * t + 3, 1)
            return c

        lax.fori_loop(1, _NCS // 2 - 1, _pipe, 0)
        _wait_g(_NCS - 2, 0)
        _compute(0)
        _scatter(_NCS - 2, 0)
        _wait_g(_NCS - 1, 1)
        _compute(1)
        _scatter(_NCS - 1, 1)

        plsc.subcore_barrier()
        pltpu.sync_copy(oacc.at[pl.ds(sid * _RPT, _RPT)],
                        out_hbm.at[p, cid, pl.ds(sid * _RPT, _RPT)])


def _spmm(f0a, f1a, st0, st1, src0, dst0, src1, dst1, exi0, exi1):
    dbuf = lambda: [
        pltpu.VMEM((_CHS, 16), _F32),
        pltpu.VMEM((_CHS, 16), _F32),
        pltpu.VMEM((_CHS, _HD), _F32),
    ]
    return pl.kernel(
        _spmm_body,
        out_type=jax.ShapeDtypeStruct((_P, _NC, _NPAD, _HD), _F32),
        mesh=_mesh(),
        compiler_params=_SC_PARAMS,
        scratch_types=[
            pltpu.VMEM((_NCS, _CHS), jnp.int32),
            pltpu.VMEM((_NCS, _CHS), jnp.int32),
        ] + dbuf() + dbuf() + [
            pltpu.VMEM((16, _HD), _F32),
            pltpu.VMEM_SHARED((_NPAD, _HD), _F32),
            pltpu.SemaphoreType.DMA,
            pltpu.SemaphoreType.DMA,
            pltpu.SemaphoreType.DMA,
            pltpu.SemaphoreType.DMA,
        ],
    )(f0a, f1a, st0, st1, src0, dst0, src1, dst1, exi0, exi1)


def _f1_body(op_ref, gt_ref, bg_ref, w1_ref, b1_ref, w2_ref, z_ref, ws_ref):
    nb = pl.program_id(1)
    operm = op_ref[0, 0] + op_ref[0, 1]
    o = jnp.dot(operm, gt_ref[...], preferred_element_type=_F32) + bg_ref[0, 0]
    z = jnp.where(o > 0, o, jnp.exp(o) - 1.0)
    z_ref[0] = z
    t = jnp.tanh(jnp.dot(z, w1_ref[...], preferred_element_type=_F32)
                 + b1_ref[...])
    wcol = jnp.sum(t * w2_ref[...], axis=1, keepdims=True)
    rows = nb * _BLK + lax.broadcasted_iota(jnp.int32, (_BLK, 1), 0)
    wcol = jnp.where(rows < _N, wcol, 0.0)
    sall = jnp.sum(wcol)

    @pl.when(nb == 0)
    def _():
        ws_ref[...] = jnp.full((1, 1, 128), sall, _F32)

    @pl.when(nb > 0)
    def _():
        ws_ref[...] = ws_ref[...] + sall


def _f1(outp, Gt, bias_gat, W_s1, b1r, w2r):
    return pl.pallas_call(
        _f1_body,
        grid=(_P, _NBLK),
        in_specs=[
            pl.BlockSpec((1, _NC, _BLK, _HD), lambda p, i: (p, 0, i, 0)),
            pl.BlockSpec((_HD, _HD), lambda p, i: (0, 0)),
            pl.BlockSpec((1, 1, _HD), lambda p, i: (p, 0, 0)),
            pl.BlockSpec((_HD, _HD), lambda p, i: (0, 0)),
            pl.BlockSpec((1, _HD), lambda p, i: (0, 0)),
            pl.BlockSpec((1, _HD), lambda p, i: (0, 0)),
        ],
        out_specs=[
            pl.BlockSpec((1, _BLK, _HD), lambda p, i: (p, i, 0)),
            pl.BlockSpec((1, 1, 128), lambda p, i: (p, 0, 0)),
        ],
        out_shape=[
            jax.ShapeDtypeStruct((_P, _NPAD, _HD), _F32),
            jax.ShapeDtypeStruct((_P, 1, 128), _F32),
        ],
    )(outp, Gt, bias_gat.reshape(_P, 1, _HD), W_s1, b1r, w2r)


def _f2_body(z_ref, ws_ref, wp_ref, bp_ref, o_ref):
    w = ws_ref[:, 0, :] / float(_N)
    m = jnp.max(w, axis=0, keepdims=True)
    ew = jnp.exp(w - m)
    beta = ew / jnp.sum(ew, axis=0, keepdims=True)
    h = z_ref[0] * beta[0:1, :] + z_ref[1] * beta[1:2, :]
    o_ref[...] = jnp.dot(h, wp_ref[...], preferred_element_type=_F32) + bp_ref[...]


def _f2(z, wsum, W_p, bpr):
    return pl.pallas_call(
        _f2_body,
        grid=(_NBLK,),
        in_specs=[
            pl.BlockSpec((_P, _BLK, _HD), lambda i: (0, i, 0)),
            pl.BlockSpec((_P, 1, 128), lambda i: (0, 0, 0)),
            pl.BlockSpec((_HD, _C), lambda i: (0, 0)),
            pl.BlockSpec((1, _C), lambda i: (0, 0)),
        ],
        out_specs=pl.BlockSpec((_BLK, _C), lambda i: (i, 0)),
        out_shape=jax.ShapeDtypeStruct((_NPAD, _C), _F32),
    )(z, wsum, W_p, bpr)


def kernel(x, W_gat, attn_l, attn_r, bias_gat, W_s1, b_s1, W_s2, W_p, b_p,
           edge_index_0, edge_index_1):
    xpad = jnp.zeros((_NPAD, _FIN), _F32).at[:_N].set(x)
    rows = jnp.arange(_HD)
    hcol = rows // _D
    AL = jnp.zeros((_P, _HD, 16), _F32)
    AL = AL.at[:, rows, hcol].set(attn_l.reshape(_P, _HD))
    AL = AL.at[:, rows, hcol + 8].set(attn_l.reshape(_P, _HD))
    AR = jnp.zeros((_P, _HD, 16), _F32)
    AR = AR.at[:, rows, hcol].set(attn_r.reshape(_P, _HD))
    AR = AR.at[:, rows, hcol + 8].set(attn_r.reshape(_P, _HD))
    cc = jnp.arange(_HD)
    ll = cc % 16
    kk = cc // 16
    gidx = (ll % 8) * _D + 2 * kk + ll // 8
    G = jnp.zeros((_HD, _HD), _F32).at[gidx, cc].set(1.0)
    Gt = G.T

    feat, L, R = _prep(xpad, W_gat, AL, AR, G)
    pad = _N + (jnp.arange(_EPAD - _E, dtype=jnp.int32) % (_NPAD - _N))
    s0f = jnp.concatenate([edge_index_0[0], pad])
    d0f = jnp.concatenate([edge_index_0[1], pad])
    s1f = jnp.concatenate([edge_index_1[0], pad])
    d1f = jnp.concatenate([edge_index_1[1], pad])

    ra = lambda a: a.reshape(_EPAD // _CHA, _CHA)
    rs = lambda a: a.reshape(_EPAD // _CHS, _CHS)

    sA0, sA1, exA0, exA1 = _attn(L[0], R[0], L[1], R[1],
                                 ra(s0f), ra(d0f), ra(s1f), ra(d1f))
    st0, st1 = _ssum(sA0, sA1)
    outp = _spmm(feat[0], feat[1], st0, st1,
                 rs(s0f), rs(d0f), rs(s1f), rs(d1f),
                 exA0, exA1)
    z, wsum = _f1(outp, Gt, bias_gat, W_s1, b_s1.reshape(1, -1),
                  W_s2.reshape(1, -1))
    out = _f2(z, wsum, W_p, b_p.reshape(1, -1))
    return out[:_N]


# speedup vs baseline: 1.2580x; 1.0040x over previous
import jax
import jax.numpy as jnp
from jax import lax
from jax.experimental import pallas as pl
from jax.experimental.pallas import tpu as pltpu
from jax.experimental.pallas import tpu_sc as plsc

_N = 10000
_E = 320000
_FIN = 128
_H = 8
_D = 16
_HD = 128
_P = 2
_C = 16

_NPAD = 10240
_NC = 2
_NS = 16
_NW = _NC * _NS
_EPW = 10240
_EPAD = _EPW * _NW
_CHA = 128
_NCA = _EPW // _CHA
_CHS = 80
_NCS = _EPW // _CHS
_RPT = _NPAD // _NS
_BLK = 1024
_NBLK = _NPAD // _BLK

_F32 = jnp.float32
_SC_PARAMS = pltpu.CompilerParams(use_tc_tiling_on_sc=False)


def _mesh():
    return plsc.VectorSubcoreMesh(core_axis_name="c", subcore_axis_name="s",
                                  num_cores=_NC, num_subcores=_NS)


def _prep_body(x_ref, w_ref, al_ref, ar_ref, g_ref, feat_ref, l_ref, r_ref):
    f = jnp.dot(x_ref[...], w_ref[0], preferred_element_type=_F32)
    feat_ref[0] = jnp.dot(f, g_ref[...], preferred_element_type=_F32)
    l_ref[0] = jnp.dot(f, al_ref[0], preferred_element_type=_F32)
    r_ref[0] = jnp.dot(f, ar_ref[0], preferred_element_type=_F32)


def _prep(xpad, W_gat, AL, AR, G):
    return pl.pallas_call(
        _prep_body,
        grid=(_P, _NBLK),
        in_specs=[
            pl.BlockSpec((_BLK, _FIN), lambda p, i: (i, 0)),
            pl.BlockSpec((1, _FIN, _HD), lambda p, i: (p, 0, 0)),
            pl.BlockSpec((1, _HD, 16), lambda p, i: (p, 0, 0)),
            pl.BlockSpec((1, _HD, 16), lambda p, i: (p, 0, 0)),
            pl.BlockSpec((_HD, _HD), lambda p, i: (0, 0)),
        ],
        out_specs=[
            pl.BlockSpec((1, _BLK, _HD), lambda p, i: (p, i, 0)),
            pl.BlockSpec((1, _BLK, 16), lambda p, i: (p, i, 0)),
            pl.BlockSpec((1, _BLK, 16), lambda p, i: (p, i, 0)),
        ],
        out_shape=[
            jax.ShapeDtypeStruct((_P, _NPAD, _HD), _F32),
            jax.ShapeDtypeStruct((_P, _NPAD, 16), _F32),
            jax.ShapeDtypeStruct((_P, _NPAD, 16), _F32),
        ],
    )(xpad, W_gat, AL, AR, G)


def _attn_body(l0, r0, l1, r1, src0, dst0, src1, dst1, s_out0, s_out1,
               ex0, ex1,
               isb, idb,
               lv0, rv0, ev0,
               lv1, rv1, ev1,
               zb, sacc0, sacc1,
               semg0, semg1, semw0, semw1, semsc0, semsc1):
    cid = lax.axis_index("c")
    sid = lax.axis_index("s")
    wid = sid * _NC + cid
    SETS = ((lv0, rv0, ev0, semg0, semw0, semsc0),
            (lv1, rv1, ev1, semg1, semw1, semsc1))

    def _zrow(i, c):
        zb[i, :] = jnp.zeros((16,), _F32)
        return c

    lax.fori_loop(0, _RPT, _zrow, 0)
    pltpu.sync_copy(zb, sacc0.at[pl.ds(sid * _RPT, _RPT)])
    pltpu.sync_copy(zb, sacc1.at[pl.ds(sid * _RPT, _RPT)])
    plsc.subcore_barrier()

    for p in range(_P):
        lt = (l0, l1)[p]
        rt = (r0, r1)[p]
        se = (src0, src1)[p]
        de = (dst0, dst1)[p]
        sacc = (sacc0, sacc1)[p]
        ext = (ex0, ex1)[p]

        pltpu.sync_copy(se.at[pl.ds(wid * _NCA, _NCA)], isb)
        pltpu.sync_copy(de.at[pl.ds(wid * _NCA, _NCA)], idb)

        def _off(k):
            return wid * _EPW + k * _CHA

        def _issue(k, b):
            lv, rv, ev, semg, semw, semsc = SETS[b]
            pltpu.async_copy(lt.at[isb.at[k]], lv, semg)
            pltpu.async_copy(rt.at[idb.at[k]], rv, semg)

        def _wait_g(k, b):
            lv, rv, ev, semg, semw, semsc = SETS[b]
            pltpu.make_async_copy(lt.at[isb.at[k]], lv, semg).wait()
            pltpu.make_async_copy(rt.at[idb.at[k]], rv, semg).wait()

        def _compute(b):
            lv, rv, ev, semg, semw, semsc = SETS[b]

            def _edge(i, cc):
                e = lv[i, :] + rv[i, :]
                e = jnp.where(e > 0, e, 0.2 * e)
                ev[i, :] = jnp.exp(e)
                return cc

            lax.fori_loop(0, _CHA, _edge, 0, unroll=8)

        def _scatter(k, b):
            lv, rv, ev, semg, semw, semsc = SETS[b]
            pltpu.async_copy(ev, ext.at[pl.ds(_off(k), _CHA)], semw)
            pltpu.sync_copy(ev, sacc.at[idb.at[k]], add=True)

        def _wait_w(k, b):
            lv, rv, ev, semg, semw, semsc = SETS[b]
            pltpu.make_async_copy(ev, ext.at[pl.ds(_off(k), _CHA)], semw).wait()

        _issue(0, 0)
        _issue(1, 1)
        _wait_g(0, 0)
        _compute(0)
        _scatter(0, 0)
        _issue(2, 0)
        _wait_g(1, 1)
        _compute(1)
        _scatter(1, 1)
        _issue(3, 1)

        def _pipe(t, c):
            _wait_g(2 * t, 0)
            _wait_w(2 * t - 2, 0)
            _compute(0)
            _scatter(2 * t, 0)
            _issue(2 * t + 2, 0)
            _wait_g(2 * t + 1, 1)
            _wait_w(2 * t - 1, 1)
            _compute(1)
            _scatter(2 * t + 1, 1)
            _issue(2 * t + 3, 1)
            return c

        lax.fori_loop(1, _NCA // 2 - 1, _pipe, 0)
        _wait_g(_NCA - 2, 0)
        _wait_w(_NCA - 4, 0)
        _compute(0)
        _scatter(_NCA - 2, 0)
        _wait_g(_NCA - 1, 1)
        _wait_w(_NCA - 3, 1)
        _compute(1)
        _scatter(_NCA - 1, 1)
        _wait_w(_NCA - 2, 0)
        _wait_w(_NCA - 1, 1)

    plsc.subcore_barrier()
    pltpu.sync_copy(sacc0.at[pl.ds(sid * _RPT, _RPT)],
                    s_out0.at[cid, pl.ds(sid * _RPT, _RPT)])
    pltpu.sync_copy(sacc1.at[pl.ds(sid * _RPT, _RPT)],
                    s_out1.at[cid, pl.ds(sid * _RPT, _RPT)])


def _attn(l0a, r0a, l1a, r1a, src0, dst0, src1, dst1):
    dbuf = lambda: [
        pltpu.VMEM((_CHA, 16), _F32),
        pltpu.VMEM((_CHA, 16), _F32),
        pltpu.VMEM((_CHA, 16), _F32),
    ]
    return pl.kernel(
        _attn_body,
        out_type=[
            jax.ShapeDtypeStruct((_NC, _NPAD, 16), _F32),
            jax.ShapeDtypeStruct((_NC, _NPAD, 16), _F32),
            jax.ShapeDtypeStruct((_EPAD, 16), _F32),
            jax.ShapeDtypeStruct((_EPAD, 16), _F32),
        ],
        mesh=_mesh(),
        compiler_params=_SC_PARAMS,
        scratch_types=[
            pltpu.VMEM((_NCA, _CHA), jnp.int32),
            pltpu.VMEM((_NCA, _CHA), jnp.int32),
        ] + dbuf() + dbuf() + [
            pltpu.VMEM((_RPT, 16), _F32),
            pltpu.VMEM_SHARED((_NPAD, 16), _F32),
            pltpu.VMEM_SHARED((_NPAD, 16), _F32),
            pltpu.SemaphoreType.DMA,
            pltpu.SemaphoreType.DMA,
            pltpu.SemaphoreType.DMA,
            pltpu.SemaphoreType.DMA,
            pltpu.SemaphoreType.DMA,
            pltpu.SemaphoreType.DMA,
        ],
    )(l0a, r0a, l1a, r1a, src0, dst0, src1, dst1)


def _ssum_body(a_ref, b_ref, oa_ref, ob_ref):
    oa_ref[...] = 1.0 / (a_ref[0] + a_ref[1] + 1e-9)
    ob_ref[...] = 1.0 / (b_ref[0] + b_ref[1] + 1e-9)


def _ssum(sA0, sA1):
    return pl.pallas_call(
        _ssum_body,
        grid=(_NBLK,),
        in_specs=[
            pl.BlockSpec((_NC, _BLK, 16), lambda i: (0, i, 0)),
            pl.BlockSpec((_NC, _BLK, 16), lambda i: (0, i, 0)),
        ],
        out_specs=[
            pl.BlockSpec((_BLK, 16), lambda i: (i, 0)),
            pl.BlockSpec((_BLK, 16), lambda i: (i, 0)),
        ],
        out_shape=[
            jax.ShapeDtypeStruct((_NPAD, 16), _F32),
            jax.ShapeDtypeStruct((_NPAD, 16), _F32),
        ],
    )(sA0, sA1)


def _spmm_body(f0, f1, st0, st1, src0, dst0, src1, dst1, exi0, exi1, out_hbm,
               isb, idb,
               ev0, sv0, fv0,
               ev1, sv1, fv1,
               zb2, oacc,
               semg0, semg1, semsc0, semsc1):
    cid = lax.axis_index("c")
    sid = lax.axis_index("s")
    wid = sid * _NC + cid
    SETS = ((ev0, sv0, fv0, semg0, semsc0),
            (ev1, sv1, fv1, semg1, semsc1))

    def _zrow(i, c):
        for j in range(8):
            zb2[i, pl.ds(16 * j, 16)] = jnp.zeros((16,), _F32)
        return c

    lax.fori_loop(0, 16, _zrow, 0)

    for p in range(_P):
        ft = (f0, f1)[p]
        st = (st0, st1)[p]
        se = (src0, src1)[p]
        de = (dst0, dst1)[p]
        ext = (exi0, exi1)[p]

        nz = _RPT // 16
        def _zacc(t, c):
            pltpu.async_copy(zb2, oacc.at[pl.ds(sid * _RPT + t * 16, 16)],
                             semg0)
            return c

        lax.fori_loop(0, nz, _zacc, 0)

        def _zwait(t, c):
            pltpu.make_async_copy(zb2, oacc.at[pl.ds(sid * _RPT, 16)],
                                  semg0).wait()
            return c

        lax.fori_loop(0, nz, _zwait, 0)
        plsc.subcore_barrier()

        pltpu.sync_copy(se.at[pl.ds(wid * _NCS, _NCS)], isb)
        pltpu.sync_copy(de.at[pl.ds(wid * _NCS, _NCS)], idb)

        def _off(k):
            return wid * _EPW + k * _CHS

        def _issue(k, b):
            ev, sv, fv, semg, semsc = SETS[b]
            pltpu.async_copy(ext.at[pl.ds(_off(k), _CHS)], ev, semg)
            pltpu.async_copy(st.at[idb.at[k]], sv, semg)
            pltpu.async_copy(ft.at[isb.at[k]], fv, semg)

        def _wait_g(k, b):
            ev, sv, fv, semg, semsc = SETS[b]
            pltpu.make_async_copy(ext.at[pl.ds(_off(k), _CHS)], ev, semg).wait()
            pltpu.make_async_copy(st.at[idb.at[k]], sv, semg).wait()
            pltpu.make_async_copy(ft.at[isb.at[k]], fv, semg).wait()

        def _compute(b):
            ev, sv, fv, semg, semsc = SETS[b]

            def _edge(i, cc):
                av = ev[i, :] * sv[i, :]
                for j in range(8):
                    fv[i, pl.ds(16 * j, 16)] = fv[i, pl.ds(16 * j, 16)] * av
                return cc

            lax.fori_loop(0, _CHS, _edge, 0, unroll=4)

        def _scatter(k, b):
            ev, sv, fv, semg, semsc = SETS[b]
            pltpu.sync_copy(fv, oacc.at[idb.at[k]], add=True)

        _issue(0, 0)
        _issue(1, 1)
        _wait_g(0, 0)
        _compute(0)
        _scatter(0, 0)
        _issue(2, 0)
        _wait_g(1, 1)
        _compute(1)
        _scatter(1, 1)
        _issue(3, 1)

        def _pipe(t, c):
            _wait_g(2 * t, 0)
            _compute(0)
            _scatter(2 * t, 0)
            _issue(2 * t + 2, 0)
            _wait_g(2 * t + 1, 1)
            _compute(1)
            _scatter(2 * t + 1, 1)
            _issue(2 * t + 3, 1)
            return c

        lax.fori_loop(1, _NCS // 2 - 1, _pipe, 0)
        _wait_g(_NCS - 2, 0)
        _compute(0)
        _scatter(_NCS - 2, 0)
        _wait_g(_NCS - 1, 1)
        _compute(1)
        _scatter(_NCS - 1, 1)

        plsc.subcore_barrier()
        pltpu.sync_copy(oacc.at[pl.ds(sid * _RPT, _RPT)],
                        out_hbm.at[p, cid, pl.ds(sid * _RPT, _RPT)])


def _spmm(f0a, f1a, st0, st1, src0, dst0, src1, dst1, exi0, exi1):
    dbuf = lambda: [
        pltpu.VMEM((_CHS, 16), _F32),
        pltpu.VMEM((_CHS, 16), _F32),
        pltpu.VMEM((_CHS, _HD), _F32),
    ]
    return pl.kernel(
        _spmm_body,
        out_type=jax.ShapeDtypeStruct((_P, _NC, _NPAD, _HD), _F32),
        mesh=_mesh(),
        compiler_params=_SC_PARAMS,
        scratch_types=[
            pltpu.VMEM((_NCS, _CHS), jnp.int32),
            pltpu.VMEM((_NCS, _CHS), jnp.int32),
        ] + dbuf() + dbuf() + [
            pltpu.VMEM((16, _HD), _F32),
            pltpu.VMEM_SHARED((_NPAD, _HD), _F32),
            pltpu.SemaphoreType.DMA,
            pltpu.SemaphoreType.DMA,
            pltpu.SemaphoreType.DMA,
            pltpu.SemaphoreType.DMA,
        ],
    )(f0a, f1a, st0, st1, src0, dst0, src1, dst1, exi0, exi1)


def _f1_body(op_ref, gt_ref, bg_ref, w1_ref, b1_ref, w2_ref, z_ref, ws_ref):
    nb = pl.program_id(1)
    operm = op_ref[0, 0] + op_ref[0, 1]
    o = jnp.dot(operm, gt_ref[...], preferred_element_type=_F32) + bg_ref[0, 0]
    z = jnp.where(o > 0, o, jnp.exp(o) - 1.0)
    z_ref[0] = z
    t = jnp.tanh(jnp.dot(z, w1_ref[...], preferred_element_type=_F32)
                 + b1_ref[...])
    wcol = jnp.sum(t * w2_ref[...], axis=1, keepdims=True)
    rows = nb * _BLK + lax.broadcasted_iota(jnp.int32, (_BLK, 1), 0)
    wcol = jnp.where(rows < _N, wcol, 0.0)
    sall = jnp.sum(wcol)

    @pl.when(nb == 0)
    def _():
        ws_ref[...] = jnp.full((1, 1, 128), sall, _F32)

    @pl.when(nb > 0)
    def _():
        ws_ref[...] = ws_ref[...] + sall


def _f1(outp, Gt, bias_gat, W_s1, b1r, w2r):
    return pl.pallas_call(
        _f1_body,
        grid=(_P, _NBLK),
        in_specs=[
            pl.BlockSpec((1, _NC, _BLK, _HD), lambda p, i: (p, 0, i, 0)),
            pl.BlockSpec((_HD, _HD), lambda p, i: (0, 0)),
            pl.BlockSpec((1, 1, _HD), lambda p, i: (p, 0, 0)),
            pl.BlockSpec((_HD, _HD), lambda p, i: (0, 0)),
            pl.BlockSpec((1, _HD), lambda p, i: (0, 0)),
            pl.BlockSpec((1, _HD), lambda p, i: (0, 0)),
        ],
        out_specs=[
            pl.BlockSpec((1, _BLK, _HD), lambda p, i: (p, i, 0)),
            pl.BlockSpec((1, 1, 128), lambda p, i: (p, 0, 0)),
        ],
        out_shape=[
            jax.ShapeDtypeStruct((_P, _NPAD, _HD), _F32),
            jax.ShapeDtypeStruct((_P, 1, 128), _F32),
        ],
    )(outp, Gt, bias_gat.reshape(_P, 1, _HD), W_s1, b1r, w2r)


def _f2_body(z_ref, ws_ref, wp_ref, bp_ref, o_ref):
    w = ws_ref[:, 0, :] / float(_N)
    m = jnp.max(w, axis=0, keepdims=True)
    ew = jnp.exp(w - m)
    beta = ew / jnp.sum(ew, axis=0, keepdims=True)
    h = z_ref[0] * beta[0:1, :] + z_ref[1] * beta[1:2, :]
    o_ref[...] = jnp.dot(h, wp_ref[...], preferred_element_type=_F32) + bp_ref[...]


def _f2(z, wsum, W_p, bpr):
    return pl.pallas_call(
        _f2_body,
        grid=(_NBLK,),
        in_specs=[
            pl.BlockSpec((_P, _BLK, _HD), lambda i: (0, i, 0)),
            pl.BlockSpec((_P, 1, 128), lambda i: (0, 0, 0)),
            pl.BlockSpec((_HD, _C), lambda i: (0, 0)),
            pl.BlockSpec((1, _C), lambda i: (0, 0)),
        ],
        out_specs=pl.BlockSpec((_BLK, _C), lambda i: (i, 0)),
        out_shape=jax.ShapeDtypeStruct((_NPAD, _C), _F32),
    )(z, wsum, W_p, bpr)


def kernel(x, W_gat, attn_l, attn_r, bias_gat, W_s1, b_s1, W_s2, W_p, b_p,
           edge_index_0, edge_index_1):
    xpad = jnp.zeros((_NPAD, _FIN), _F32).at[:_N].set(x)
    rows = jnp.arange(_HD)
    hcol = rows // _D
    AL = jnp.zeros((_P, _HD, 16), _F32)
    AL = AL.at[:, rows, hcol].set(attn_l.reshape(_P, _HD))
    AL = AL.at[:, rows, hcol + 8].set(attn_l.reshape(_P, _HD))
    AR = jnp.zeros((_P, _HD, 16), _F32)
    AR = AR.at[:, rows, hcol].set(attn_r.reshape(_P, _HD))
    AR = AR.at[:, rows, hcol + 8].set(attn_r.reshape(_P, _HD))
    cc = jnp.arange(_HD)
    ll = cc % 16
    kk = cc // 16
    gidx = (ll % 8) * _D + 2 * kk + ll // 8
    G = jnp.zeros((_HD, _HD), _F32).at[gidx, cc].set(1.0)
    Gt = G.T

    feat, L, R = _prep(xpad, W_gat, AL, AR, G)
    pad = _N + (jnp.arange(_EPAD - _E, dtype=jnp.int32) % (_NPAD - _N))
    s0f = jnp.concatenate([edge_index_0[0], pad])
    d0f = jnp.concatenate([edge_index_0[1], pad])
    s1f = jnp.concatenate([edge_index_1[0], pad])
    d1f = jnp.concatenate([edge_index_1[1], pad])

    ra = lambda a: a.reshape(_EPAD // _CHA, _CHA)
    rs = lambda a: a.reshape(_EPAD // _CHS, _CHS)

    sA0, sA1, exA0, exA1 = _attn(L[0], R[0], L[1], R[1],
                                 ra(s0f), ra(d0f), ra(s1f), ra(d1f))
    st0, st1 = _ssum(sA0, sA1)
    outp = _spmm(feat[0], feat[1], st0, st1,
                 rs(s0f), rs(d0f), rs(s1f), rs(d1f),
                 exA0, exA1)
    z, wsum = _f1(outp, Gt, bias_gat, W_s1, b_s1.reshape(1, -1),
                  W_s2.reshape(1, -1))
    out = _f2(z, wsum, W_p, b_p.reshape(1, -1))
    return out[:_N]
